# baseline (speedup 1.0000x reference)
import functools

import jax
import jax.numpy as jnp
from jax import lax
from jax.experimental import pallas as pl
from jax.experimental.pallas import tpu as pltpu
from jax.experimental.pallas import tpu_sc as plsc

N = 10000
E = 320000
D = 128
NT = 2
NC = 2
NS = 16
NW = NC * NS
CHUNK = 128
E_PAD = 327680
EPW = E_PAD // NW
NSLOT = 2
NSTAGE = 2
STAGE = EPW // NSTAGE
CH_PER_STAGE = STAGE // CHUNK
N_PAD = 10240
ROWS_PER_TILE = N_PAD // NS

MM_BLK = 2000
GRU_BLK = 1000


def _hw_body(h_ref, wt_ref, out_ref):
    out_ref[0] = jnp.dot(h_ref[...], wt_ref[0],
                         preferred_element_type=jnp.float32)
    out_ref[1] = jnp.dot(h_ref[...], wt_ref[1],
                         preferred_element_type=jnp.float32)


def _hw_table(h, Wt):
    nb = N // MM_BLK
    return pl.pallas_call(
        _hw_body,
        grid=(nb,),
        in_specs=[
            pl.BlockSpec((MM_BLK, D), lambda i: (i, 0)),
            pl.BlockSpec((NT, D, D), lambda i: (0, 0, 0)),
        ],
        out_specs=pl.BlockSpec((NT, MM_BLK, D), lambda i: (0, i, 0)),
        out_shape=jax.ShapeDtypeStruct((NT, N, D), jnp.float32),
    )(h, Wt)


def _sc_scatter_body(table_hbm, gidx_hbm, dst_hbm, zeros_hbm, out_hbm,
                     gidx_st, dst_st, gv0, gv1, dv0, dv1, rows_v, acc_sh,
                     sem0, sem1):
    c = lax.axis_index("c")
    s = lax.axis_index("s")
    sems = (sem0, sem1)
    gvs = (gv0, gv1)
    dvs = (dv0, dv1)

    wid = c * NS + s

    pltpu.sync_copy(zeros_hbm, acc_sh.at[pl.ds(s * ROWS_PER_TILE, ROWS_PER_TILE)])
    plsc.subcore_barrier()

    def issue(k, slot):
        for j in range(CHUNK // 16):
            sl = pl.ds(j * 16, 16)
            st = pl.ds(k * CHUNK + j * 16, 16)
            gvs[slot][sl] = gidx_st[st]
            dvs[slot][sl] = dst_st[st]
        pltpu.async_copy(table_hbm.at[gvs[slot]], rows_v.at[slot], sems[slot])

    def wait_gather(slot):
        pltpu.make_async_copy(table_hbm.at[pl.ds(0, CHUNK)], rows_v.at[slot],
                              sems[slot]).wait()

    def process_stage(base):
        pltpu.sync_copy(gidx_hbm.at[pl.ds(base, STAGE)], gidx_st)
        pltpu.sync_copy(dst_hbm.at[pl.ds(base, STAGE)], dst_st)

        for b in range(NSLOT):
            issue(b, b)

        def step(t, carry):
            for b in range(NSLOT):
                k = t * NSLOT + b
                wait_gather(b)
                pltpu.sync_copy(rows_v.at[b], acc_sh.at[dvs[b]], add=True)

                @pl.when(t < CH_PER_STAGE // NSLOT - 1)
                def _():
                    issue(k + NSLOT, b)

            return carry

        lax.fori_loop(0, CH_PER_STAGE // NSLOT, step, 0)

    for st in range(NSTAGE):
        process_stage(wid * EPW + st * STAGE)

    plsc.subcore_barrier()

    sl = pl.ds(s * ROWS_PER_TILE, ROWS_PER_TILE)
    pltpu.sync_copy(acc_sh.at[sl], out_hbm.at[c].at[sl])


@functools.cache
def _get_sc_scatter():
    return pl.kernel(
        _sc_scatter_body,
        out_type=jax.ShapeDtypeStruct((NC, N_PAD, D), jnp.float32),
        mesh=plsc.VectorSubcoreMesh(core_axis_name="c", subcore_axis_name="s"),
        scratch_types=(
            [pltpu.VMEM((STAGE,), jnp.int32),
             pltpu.VMEM((STAGE,), jnp.int32)] +
            [pltpu.VMEM((CHUNK,), jnp.int32)
             for _ in range(NSLOT)] +
            [pltpu.VMEM((CHUNK,), jnp.int32)
             for _ in range(NSLOT)] +
            [pltpu.VMEM((NSLOT, CHUNK, D), jnp.float32),
             pltpu.VMEM_SHARED((N_PAD, D), jnp.float32)] +
            [pltpu.SemaphoreType.DMA for _ in range(NSLOT)]
        ),
    )


def _sc_scatter(table, gidx_p, dst_p, zeros):
    return _get_sc_scatter()(table, gidx_p, dst_p, zeros)


def _sigmoid(x):
    return 1.0 / (1.0 + jnp.exp(-x))


def _gru_math(p_ref, h_ref, wih_ref, whh_ref, bih_ref, bhh_ref):
    a = p_ref[0] + p_ref[1]
    h = h_ref[...]
    gi = jnp.dot(a, wih_ref[...], preferred_element_type=jnp.float32) + bih_ref[...]
    gh = jnp.dot(h, whh_ref[...], preferred_element_type=jnp.float32) + bhh_ref[...]
    i_r, i_z, i_n = gi[:, :D], gi[:, D:2 * D], gi[:, 2 * D:]
    h_r, h_z, h_n = gh[:, :D], gh[:, D:2 * D], gh[:, 2 * D:]
    r = _sigmoid(i_r + h_r)
    z = _sigmoid(i_z + h_z)
    n = jnp.tanh(i_n + r * h_n)
    return (1.0 - z) * n + z * h


def _gru_body(p_ref, h_ref, wih_ref, whh_ref, bih_ref, bhh_ref, out_ref):
    out_ref[...] = _gru_math(p_ref, h_ref, wih_ref, whh_ref, bih_ref, bhh_ref)


def _gru_hw_body(p_ref, h_ref, wih_ref, whh_ref, bih_ref, bhh_ref, wt_ref,
                 hn_ref, hw_ref):
    hn = _gru_math(p_ref, h_ref, wih_ref, whh_ref, bih_ref, bhh_ref)
    hn_ref[...] = hn
    hw_ref[0] = jnp.dot(hn, wt_ref[0], preferred_element_type=jnp.float32)
    hw_ref[1] = jnp.dot(hn, wt_ref[1], preferred_element_type=jnp.float32)


_GRU_SPECS = [
    pl.BlockSpec((NC, GRU_BLK, D), lambda i: (0, i, 0)),
    pl.BlockSpec((GRU_BLK, D), lambda i: (i, 0)),
    pl.BlockSpec((D, 3 * D), lambda i: (0, 0)),
    pl.BlockSpec((D, 3 * D), lambda i: (0, 0)),
    pl.BlockSpec((1, 3 * D), lambda i: (0, 0)),
    pl.BlockSpec((1, 3 * D), lambda i: (0, 0)),
]


def _gru(partials, h, wih_t, whh_t, bih, bhh):
    nb = N // GRU_BLK
    return pl.pallas_call(
        _gru_body,
        grid=(nb,),
        in_specs=_GRU_SPECS,
        out_specs=pl.BlockSpec((GRU_BLK, D), lambda i: (i, 0)),
        out_shape=jax.ShapeDtypeStruct((N, D), jnp.float32),
    )(partials, h, wih_t, whh_t, bih, bhh)


def _gru_hw(partials, h, wih_t, whh_t, bih, bhh, Wt):
    nb = N // GRU_BLK
    return pl.pallas_call(
        _gru_hw_body,
        grid=(nb,),
        in_specs=_GRU_SPECS + [pl.BlockSpec((NT, D, D), lambda i: (0, 0, 0))],
        out_specs=[
            pl.BlockSpec((GRU_BLK, D), lambda i: (i, 0)),
            pl.BlockSpec((NT, GRU_BLK, D), lambda i: (0, i, 0)),
        ],
        out_shape=[
            jax.ShapeDtypeStruct((N, D), jnp.float32),
            jax.ShapeDtypeStruct((NT, N, D), jnp.float32),
        ],
    )(partials, h, wih_t, whh_t, bih, bhh, Wt)


def kernel(feats, edge_index, etypes,
           W1, w_ih1, w_hh1, b_ih1, b_hh1,
           W2, w_ih2, w_hh2, b_ih2, b_hh2):
    src = edge_index[0]
    dst = edge_index[1]
    gidx = etypes * N + src
    pad = E_PAD - E
    pad_gidx = jnp.arange(pad, dtype=jnp.int32) % (NT * N)
    gidx_p = jnp.concatenate([gidx, pad_gidx])
    pad_dst = N + (jnp.arange(pad, dtype=jnp.int32) % (N_PAD - N))
    dst_p = jnp.concatenate([dst, pad_dst])
    zeros = jnp.zeros((ROWS_PER_TILE, D), jnp.float32)

    params = []
    for (W, wih, whh, bih, bhh) in (
            (W1, w_ih1, w_hh1, b_ih1, b_hh1),
            (W2, w_ih2, w_hh2, b_ih2, b_hh2)):
        Wt = jnp.swapaxes(W, 1, 2)
        params.append((Wt, wih.T, whh.T, bih[None, :], bhh[None, :]))

    h = feats
    outs = []
    table = _hw_table(h, params[0][0])
    for step in range(4):
        layer = step // 2
        Wt, wih_t, whh_t, bih2, bhh2 = params[layer]
        partials = _sc_scatter(table.reshape(NT * N, D), gidx_p, dst_p, zeros)
        if step < 3:
            next_wt = params[(step + 1) // 2][0]
            h, table = _gru_hw(partials, h, wih_t, whh_t, bih2, bhh2, next_wt)
        else:
            h = _gru(partials, h, wih_t, whh_t, bih2, bhh2)
        if step % 2 == 1:
            outs.append(h)
    return jnp.stack(outs, axis=0)


# speedup vs baseline: 1.1058x; 1.1058x over previous
import functools

import jax
import jax.numpy as jnp
from jax import lax
from jax.experimental import pallas as pl
from jax.experimental.pallas import tpu as pltpu
from jax.experimental.pallas import tpu_sc as plsc

N = 10000
E = 320000
D = 128
NT = 2
NC = 2
NS = 16
NW = NC * NS
CHUNK = 64
E_PAD = 327680
EPW = E_PAD // NW
NSLOT = 4
NSTAGE = 2
STAGE = EPW // NSTAGE
CH_PER_STAGE = STAGE // CHUNK
N_PAD = 10240
ROWS_PER_TILE = N_PAD // NS

MM_BLK = 2000
GRU_BLK = 1000


def _hw_body(h_ref, wt_ref, out_ref):
    out_ref[0] = jnp.dot(h_ref[...], wt_ref[0],
                         preferred_element_type=jnp.float32)
    out_ref[1] = jnp.dot(h_ref[...], wt_ref[1],
                         preferred_element_type=jnp.float32)


def _hw_table(h, Wt):
    nb = N // MM_BLK
    return pl.pallas_call(
        _hw_body,
        grid=(nb,),
        in_specs=[
            pl.BlockSpec((MM_BLK, D), lambda i: (i, 0)),
            pl.BlockSpec((NT, D, D), lambda i: (0, 0, 0)),
        ],
        out_specs=pl.BlockSpec((NT, MM_BLK, D), lambda i: (0, i, 0)),
        out_shape=jax.ShapeDtypeStruct((NT, N, D), jnp.float32),
    )(h, Wt)


def _sc_scatter_body(table_hbm, gidx_hbm, dst_hbm, zeros_hbm, out_hbm,
                     gidx_st, dst_st, gv0, gv1, gv2, gv3,
                     dv0, dv1, dv2, dv3, rows_v, acc_sh,
                     sem0, sem1, sem2, sem3):
    c = lax.axis_index("c")
    s = lax.axis_index("s")
    sems = (sem0, sem1, sem2, sem3)
    gvs = (gv0, gv1, gv2, gv3)
    dvs = (dv0, dv1, dv2, dv3)

    wid = c * NS + s

    pltpu.sync_copy(zeros_hbm, acc_sh.at[pl.ds(s * ROWS_PER_TILE, ROWS_PER_TILE)])
    plsc.subcore_barrier()

    def issue(k, slot):
        for j in range(CHUNK // 16):
            sl = pl.ds(j * 16, 16)
            st = pl.ds(k * CHUNK + j * 16, 16)
            gvs[slot][sl] = gidx_st[st]
            dvs[slot][sl] = dst_st[st]
        pltpu.async_copy(table_hbm.at[gvs[slot]], rows_v.at[slot], sems[slot])

    def wait_gather(slot):
        pltpu.make_async_copy(table_hbm.at[pl.ds(0, CHUNK)], rows_v.at[slot],
                              sems[slot]).wait()

    def process_stage(base):
        pltpu.sync_copy(gidx_hbm.at[pl.ds(base, STAGE)], gidx_st)
        pltpu.sync_copy(dst_hbm.at[pl.ds(base, STAGE)], dst_st)

        for b in range(NSLOT):
            issue(b, b)

        def step(t, carry):
            for b in range(NSLOT):
                k = t * NSLOT + b
                wait_gather(b)
                pltpu.sync_copy(rows_v.at[b], acc_sh.at[dvs[b]], add=True)

                @pl.when(t < CH_PER_STAGE // NSLOT - 1)
                def _():
                    issue(k + NSLOT, b)

            return carry

        lax.fori_loop(0, CH_PER_STAGE // NSLOT, step, 0)

    for st in range(NSTAGE):
        process_stage(wid * EPW + st * STAGE)

    plsc.subcore_barrier()

    sl = pl.ds(s * ROWS_PER_TILE, ROWS_PER_TILE)
    pltpu.sync_copy(acc_sh.at[sl], out_hbm.at[c].at[sl])


@functools.cache
def _get_sc_scatter():
    return pl.kernel(
        _sc_scatter_body,
        out_type=jax.ShapeDtypeStruct((NC, N_PAD, D), jnp.float32),
        mesh=plsc.VectorSubcoreMesh(core_axis_name="c", subcore_axis_name="s"),
        scratch_types=(
            [pltpu.VMEM((STAGE,), jnp.int32),
             pltpu.VMEM((STAGE,), jnp.int32)] +
            [pltpu.VMEM((CHUNK,), jnp.int32)
             for _ in range(NSLOT)] +
            [pltpu.VMEM((CHUNK,), jnp.int32)
             for _ in range(NSLOT)] +
            [pltpu.VMEM((NSLOT, CHUNK, D), jnp.float32),
             pltpu.VMEM_SHARED((N_PAD, D), jnp.float32)] +
            [pltpu.SemaphoreType.DMA for _ in range(NSLOT)]
        ),
    )


def _sc_scatter(table, gidx_p, dst_p, zeros):
    return _get_sc_scatter()(table, gidx_p, dst_p, zeros)


def _sigmoid(x):
    return 1.0 / (1.0 + jnp.exp(-x))


def _gru_math(p_ref, h_ref, wih_ref, whh_ref, bih_ref, bhh_ref):
    a = p_ref[0] + p_ref[1]
    h = h_ref[...]
    gi = jnp.dot(a, wih_ref[...], preferred_element_type=jnp.float32) + bih_ref[...]
    gh = jnp.dot(h, whh_ref[...], preferred_element_type=jnp.float32) + bhh_ref[...]
    i_r, i_z, i_n = gi[:, :D], gi[:, D:2 * D], gi[:, 2 * D:]
    h_r, h_z, h_n = gh[:, :D], gh[:, D:2 * D], gh[:, 2 * D:]
    r = _sigmoid(i_r + h_r)
    z = _sigmoid(i_z + h_z)
    n = jnp.tanh(i_n + r * h_n)
    return (1.0 - z) * n + z * h


def _gru_body(p_ref, h_ref, wih_ref, whh_ref, bih_ref, bhh_ref, out_ref):
    out_ref[...] = _gru_math(p_ref, h_ref, wih_ref, whh_ref, bih_ref, bhh_ref)


def _gru_hw_body(p_ref, h_ref, wih_ref, whh_ref, bih_ref, bhh_ref, wt_ref,
                 hn_ref, hw_ref):
    hn = _gru_math(p_ref, h_ref, wih_ref, whh_ref, bih_ref, bhh_ref)
    hn_ref[...] = hn
    hw_ref[0] = jnp.dot(hn, wt_ref[0], preferred_element_type=jnp.float32)
    hw_ref[1] = jnp.dot(hn, wt_ref[1], preferred_element_type=jnp.float32)


_GRU_SPECS = [
    pl.BlockSpec((NC, GRU_BLK, D), lambda i: (0, i, 0)),
    pl.BlockSpec((GRU_BLK, D), lambda i: (i, 0)),
    pl.BlockSpec((D, 3 * D), lambda i: (0, 0)),
    pl.BlockSpec((D, 3 * D), lambda i: (0, 0)),
    pl.BlockSpec((1, 3 * D), lambda i: (0, 0)),
    pl.BlockSpec((1, 3 * D), lambda i: (0, 0)),
]


def _gru(partials, h, wih_t, whh_t, bih, bhh):
    nb = N // GRU_BLK
    return pl.pallas_call(
        _gru_body,
        grid=(nb,),
        in_specs=_GRU_SPECS,
        out_specs=pl.BlockSpec((GRU_BLK, D), lambda i: (i, 0)),
        out_shape=jax.ShapeDtypeStruct((N, D), jnp.float32),
    )(partials, h, wih_t, whh_t, bih, bhh)


def _gru_hw(partials, h, wih_t, whh_t, bih, bhh, Wt):
    nb = N // GRU_BLK
    return pl.pallas_call(
        _gru_hw_body,
        grid=(nb,),
        in_specs=_GRU_SPECS + [pl.BlockSpec((NT, D, D), lambda i: (0, 0, 0))],
        out_specs=[
            pl.BlockSpec((GRU_BLK, D), lambda i: (i, 0)),
            pl.BlockSpec((NT, GRU_BLK, D), lambda i: (0, i, 0)),
        ],
        out_shape=[
            jax.ShapeDtypeStruct((N, D), jnp.float32),
            jax.ShapeDtypeStruct((NT, N, D), jnp.float32),
        ],
    )(partials, h, wih_t, whh_t, bih, bhh, Wt)


def kernel(feats, edge_index, etypes,
           W1, w_ih1, w_hh1, b_ih1, b_hh1,
           W2, w_ih2, w_hh2, b_ih2, b_hh2):
    src = edge_index[0]
    dst = edge_index[1]
    gidx = etypes * N + src
    pad = E_PAD - E
    pad_gidx = jnp.arange(pad, dtype=jnp.int32) % (NT * N)
    gidx_p = jnp.concatenate([gidx, pad_gidx])
    pad_dst = N + (jnp.arange(pad, dtype=jnp.int32) % (N_PAD - N))
    dst_p = jnp.concatenate([dst, pad_dst])
    zeros = jnp.zeros((ROWS_PER_TILE, D), jnp.float32)

    params = []
    for (W, wih, whh, bih, bhh) in (
            (W1, w_ih1, w_hh1, b_ih1, b_hh1),
            (W2, w_ih2, w_hh2, b_ih2, b_hh2)):
        Wt = jnp.swapaxes(W, 1, 2)
        params.append((Wt, wih.T, whh.T, bih[None, :], bhh[None, :]))

    h = feats
    outs = []
    table = _hw_table(h, params[0][0])
    for step in range(4):
        layer = step // 2
        Wt, wih_t, whh_t, bih2, bhh2 = params[layer]
        partials = _sc_scatter(table.reshape(NT * N, D), gidx_p, dst_p, zeros)
        if step < 3:
            next_wt = params[(step + 1) // 2][0]
            h, table = _gru_hw(partials, h, wih_t, whh_t, bih2, bhh2, next_wt)
        else:
            h = _gru(partials, h, wih_t, whh_t, bih2, bhh2)
        if step % 2 == 1:
            outs.append(h)
    return jnp.stack(outs, axis=0)
